# baseline (speedup 1.0000x reference)
import functools

import jax
import jax.numpy as jnp
from jax import lax
from jax.experimental import pallas as pl
from jax.experimental.pallas import tpu as pltpu
from jax.experimental.pallas import tpu_sc as plsc

N = 10000
E = 320000
D = 128
DE = 16

NC = 2
NS = 16
NW = NC * NS
EPW = E // NW
CHUNK = 80
NCHUNK = EPW // CHUNK
LANES = 16

_mesh = plsc.VectorSubcoreMesh(core_axis_name="c", subcore_axis_name="s")

DAB = D + DE


def _copy_out(sh_ref, out_hbm, cid, sid):
    rps = (N // NS) // 8 * 8

    @pl.when(sid < NS - 1)
    def _():
        r0 = sid * rps
        pltpu.sync_copy(sh_ref.at[pl.ds(r0, rps)],
                        out_hbm.at[cid].at[pl.ds(r0, rps)])

    @pl.when(sid == NS - 1)
    def _():
        r0 = (NS - 1) * rps
        pltpu.sync_copy(sh_ref.at[pl.ds(r0, N - r0)],
                        out_hbm.at[cid].at[pl.ds(r0, N - r0)])


def _p1a_body(x_hbm, src_hbm, dst_hbm, t_hbm, za_hbm,
              apart_hbm,
              a_sh, src_v, dst_v, t_v, rows_v, sem):
    cid = lax.axis_index("c")
    sid = lax.axis_index("s")
    wid = cid * NS + sid
    base = wid * EPW

    @pl.when(sid == 0)
    def _():
        pltpu.sync_copy(za_hbm, a_sh)
    plsc.subcore_barrier()

    def chunk_body(i, carry):
        off = base + i * CHUNK
        pltpu.sync_copy(src_hbm.at[pl.ds(off, CHUNK)], src_v)
        pltpu.sync_copy(dst_hbm.at[pl.ds(off, CHUNK)], dst_v)
        pltpu.sync_copy(t_hbm.at[pl.ds(off, CHUNK)], t_v)
        pltpu.async_copy(x_hbm.at[src_v], rows_v, sem).wait()

        def grp_body(g, carry2):
            tv = t_v[pl.ds(g * LANES, LANES)]
            s_vec = 1.0 / (1.0 + jnp.exp(-tv))
            for k in range(LANES):
                e = g * LANES + k
                sc = s_vec[k]
                for j in range(D // LANES):
                    sl = pl.ds(j * LANES, LANES)
                    rows_v[e, sl] = rows_v[e, sl] * sc
            return carry2
        lax.fori_loop(0, CHUNK // LANES, grp_body, 0)

        pltpu.sync_copy(rows_v, a_sh.at[dst_v], add=True)
        return carry
    lax.fori_loop(0, NCHUNK, chunk_body, 0)

    plsc.subcore_barrier()
    _copy_out(a_sh, apart_hbm, cid, sid)


_p1a = pl.kernel(
    _p1a_body,
    out_type=jax.ShapeDtypeStruct((NC, N, D), jnp.float32),
    mesh=_mesh,
    scratch_types=[
        pltpu.VMEM_SHARED((N, D), jnp.float32),
        pltpu.VMEM((CHUNK,), jnp.int32),
        pltpu.VMEM((CHUNK,), jnp.int32),
        pltpu.VMEM((CHUNK,), jnp.float32),
        pltpu.VMEM((CHUNK, D), jnp.float32),
        pltpu.SemaphoreType.DMA,
    ],
)


def _p1b_body(dst_hbm, smsg_hbm, zb_hbm,
              bpart_hbm,
              b_sh, dst_v, mrow_v, sem):
    cid = lax.axis_index("c")
    sid = lax.axis_index("s")
    wid = cid * NS + sid
    base = wid * EPW

    @pl.when(sid == 0)
    def _():
        pltpu.sync_copy(zb_hbm, b_sh)
    plsc.subcore_barrier()

    def chunk_body(i, carry):
        off = base + i * CHUNK
        pltpu.sync_copy(dst_hbm.at[pl.ds(off, CHUNK)], dst_v)
        pltpu.sync_copy(smsg_hbm.at[pl.ds(off, CHUNK), :], mrow_v)
        pltpu.sync_copy(mrow_v, b_sh.at[dst_v], add=True)
        return carry
    lax.fori_loop(0, NCHUNK, chunk_body, 0)

    plsc.subcore_barrier()
    _copy_out(b_sh, bpart_hbm, cid, sid)


_p1b = pl.kernel(
    _p1b_body,
    out_type=jax.ShapeDtypeStruct((NC, N, DE), jnp.float32),
    mesh=_mesh,
    scratch_types=[
        pltpu.VMEM_SHARED((N, DE), jnp.float32),
        pltpu.VMEM((CHUNK,), jnp.int32),
        pltpu.VMEM((CHUNK, DE), jnp.float32),
        pltpu.SemaphoreType.DMA,
    ],
)


_SB = 8000

def _smsg_body(t_ref, msg_ref, out_ref):
    out_ref[...] = msg_ref[...] * jax.nn.sigmoid(t_ref[...])


def _smsg_call(t, msg):
    return pl.pallas_call(
        _smsg_body,
        grid=(E // _SB,),
        in_specs=[
            pl.BlockSpec((_SB, 1), lambda i: (i, 0)),
            pl.BlockSpec((_SB, DE), lambda i: (i, 0)),
        ],
        out_specs=pl.BlockSpec((_SB, DE), lambda i: (i, 0)),
        out_shape=jax.ShapeDtypeStruct((E, DE), jnp.float32),
    )(t.reshape(E, 1), msg)


def _h_body(ap_ref, bp_ref, x_ref, wmsg_ref, wself_ref, wupd_ref, h_ref):
    a = ap_ref[0] + ap_ref[1]
    b = bp_ref[0] + bp_ref[1]
    agg = jnp.dot(a, wmsg_ref[0:D, :], preferred_element_type=jnp.float32)
    agg += jnp.dot(b, wmsg_ref[D:DAB, :], preferred_element_type=jnp.float32)
    acc = jnp.dot(agg, wupd_ref[...], preferred_element_type=jnp.float32)
    acc += jnp.dot(x_ref[...], wself_ref[...], preferred_element_type=jnp.float32)
    h_ref[...] = jnp.maximum(acc, 0.0)


_HR = 1000

def _h_call(apart, bpart, x, w_msg, w_self, w_upd):
    return pl.pallas_call(
        _h_body,
        grid=(N // _HR,),
        in_specs=[
            pl.BlockSpec((NC, _HR, D), lambda i: (0, i, 0)),
            pl.BlockSpec((NC, _HR, DE), lambda i: (0, i, 0)),
            pl.BlockSpec((_HR, D), lambda i: (i, 0)),
            pl.BlockSpec((DAB, D), lambda i: (0, 0)),
            pl.BlockSpec((D, D), lambda i: (0, 0)),
            pl.BlockSpec((D, D), lambda i: (0, 0)),
        ],
        out_specs=pl.BlockSpec((_HR, D), lambda i: (i, 0)),
        out_shape=jax.ShapeDtypeStruct((N, D), jnp.float32),
    )(apart, bpart, x, w_msg, w_self, w_upd)


def _p3_body(h_hbm, src_hbm, dst_hbm, pd_hbm,
             src_v, dst_v, hs_v, hd_v, pd_v, sem):
    cid = lax.axis_index("c")
    sid = lax.axis_index("s")
    wid = cid * NS + sid
    base = wid * EPW

    def chunk_body(i, carry):
        off = base + i * CHUNK
        pltpu.sync_copy(src_hbm.at[pl.ds(off, CHUNK)], src_v)
        pltpu.sync_copy(dst_hbm.at[pl.ds(off, CHUNK)], dst_v)
        cp1 = pltpu.async_copy(h_hbm.at[src_v], hs_v, sem)
        cp2 = pltpu.async_copy(h_hbm.at[dst_v], hd_v, sem)
        cp1.wait()
        cp2.wait()

        def edge_body(e, carry2):
            acc = hs_v[e, pl.ds(0, LANES)] * hd_v[e, pl.ds(0, LANES)]
            for j in range(1, D // LANES):
                sl = pl.ds(j * LANES, LANES)
                acc = acc + hs_v[e, sl] * hd_v[e, sl]
            pd_v[e, :] = acc
            return carry2
        lax.fori_loop(0, CHUNK, edge_body, 0)

        pltpu.sync_copy(pd_v, pd_hbm.at[pl.ds(off, CHUNK), :])
        return carry
    lax.fori_loop(0, NCHUNK, chunk_body, 0)


_p3 = pl.kernel(
    _p3_body,
    out_type=jax.ShapeDtypeStruct((E, LANES), jnp.float32),
    mesh=_mesh,
    scratch_types=[
        pltpu.VMEM((CHUNK,), jnp.int32),
        pltpu.VMEM((CHUNK,), jnp.int32),
        pltpu.VMEM((CHUNK, D), jnp.float32),
        pltpu.VMEM((CHUNK, D), jnp.float32),
        pltpu.VMEM((CHUNK, LANES), jnp.float32),
        pltpu.SemaphoreType.DMA,
    ],
)


_EB = 8000

def _loss_body(pd_ref, out_ref):
    i = pl.program_id(0)
    logits = jnp.sum(pd_ref[...], axis=1)
    part = jnp.sum(jax.nn.softplus(-logits)) * (1.0 / E)

    @pl.when(i == 0)
    def _():
        out_ref[0, 0] = part

    @pl.when(i > 0)
    def _():
        out_ref[0, 0] += part


def _loss_call(pdot):
    return pl.pallas_call(
        _loss_body,
        grid=(E // _EB,),
        in_specs=[pl.BlockSpec((_EB, LANES), lambda i: (i, 0))],
        out_specs=pl.BlockSpec(memory_space=pltpu.SMEM),
        out_shape=jax.ShapeDtypeStruct((1, 1), jnp.float32),
    )(pdot)


def kernel(x, edge_index, t, msg, edge_type, W_msg, W_self, W_upd):
    del edge_type
    src = edge_index[0]
    dst = edge_index[1]
    za = jnp.zeros((N, D), jnp.float32)
    zb = jnp.zeros((N, DE), jnp.float32)
    apart = _p1a(x, src, dst, t, za)
    bpart = _p1b(dst, _smsg_call(t, msg), zb)
    h = _h_call(apart, bpart, x, W_msg, W_self, W_upd)
    pdot = _p3(h, src, dst)
    loss = _loss_call(pdot)
    return loss.reshape((1,))


# speedup vs baseline: 1.4615x; 1.4615x over previous
import functools

import jax
import jax.numpy as jnp
from jax import lax
from jax.experimental import pallas as pl
from jax.experimental.pallas import tpu as pltpu
from jax.experimental.pallas import tpu_sc as plsc

N = 10000
E = 320000
D = 128
DE = 16

NC = 2
NS = 16
NW = NC * NS
EPW = E // NW
CHUNK = 80
NCHUNK = EPW // CHUNK
LANES = 16

_mesh = plsc.VectorSubcoreMesh(core_axis_name="c", subcore_axis_name="s")

DAB = D + DE


def _copy_out(sh_ref, out_hbm, cid, sid):
    rps = (N // NS) // 8 * 8

    @pl.when(sid < NS - 1)
    def _():
        r0 = sid * rps
        pltpu.sync_copy(sh_ref.at[pl.ds(r0, rps)],
                        out_hbm.at[cid].at[pl.ds(r0, rps)])

    @pl.when(sid == NS - 1)
    def _():
        r0 = (NS - 1) * rps
        pltpu.sync_copy(sh_ref.at[pl.ds(r0, N - r0)],
                        out_hbm.at[cid].at[pl.ds(r0, N - r0)])


def _p1a_body(x_hbm, src_hbm, dst_hbm, t_hbm, za_hbm,
              apart_hbm,
              a_sh, src0_v, src1_v, dst0_v, dst1_v, t0_v, t1_v,
              rows0_v, rows1_v, sem0, sem1):
    cid = lax.axis_index("c")
    sid = lax.axis_index("s")
    wid = cid * NS + sid
    base = wid * EPW

    @pl.when(sid == 0)
    def _():
        pltpu.sync_copy(za_hbm, a_sh)
    plsc.subcore_barrier()

    def load_and_gather(c, src_v, dst_v, t_v, rows_v, sem):
        off = base + c * CHUNK
        pltpu.sync_copy(src_hbm.at[pl.ds(off, CHUNK)], src_v)
        pltpu.sync_copy(dst_hbm.at[pl.ds(off, CHUNK)], dst_v)
        pltpu.sync_copy(t_hbm.at[pl.ds(off, CHUNK)], t_v)
        pltpu.async_copy(x_hbm.at[src_v], rows_v, sem)

    def drain_gather(src_v, rows_v, sem):
        pltpu.make_async_copy(x_hbm.at[src_v], rows_v, sem).wait()

    def scale_and_scatter(t_v, dst_v, rows_v):
        def grp_body(g, carry2):
            tv = t_v[pl.ds(g * LANES, LANES)]
            s_vec = 1.0 / (1.0 + jnp.exp(-tv))
            for k in range(LANES):
                e = g * LANES + k
                sc = s_vec[k]
                for j in range(D // LANES):
                    sl = pl.ds(j * LANES, LANES)
                    rows_v[e, sl] = rows_v[e, sl] * sc
            return carry2
        lax.fori_loop(0, CHUNK // LANES, grp_body, 0)
        pltpu.sync_copy(rows_v, a_sh.at[dst_v], add=True)

    load_and_gather(0, src0_v, dst0_v, t0_v, rows0_v, sem0)

    def pair_body(i, carry):
        c0 = 2 * i
        load_and_gather(c0 + 1, src1_v, dst1_v, t1_v, rows1_v, sem1)
        drain_gather(src0_v, rows0_v, sem0)
        scale_and_scatter(t0_v, dst0_v, rows0_v)
        load_and_gather(c0 + 2, src0_v, dst0_v, t0_v, rows0_v, sem0)
        drain_gather(src1_v, rows1_v, sem1)
        scale_and_scatter(t1_v, dst1_v, rows1_v)
        return carry
    lax.fori_loop(0, (NCHUNK - 1) // 2, pair_body, 0)

    drain_gather(src0_v, rows0_v, sem0)
    scale_and_scatter(t0_v, dst0_v, rows0_v)

    plsc.subcore_barrier()
    _copy_out(a_sh, apart_hbm, cid, sid)


_p1a = pl.kernel(
    _p1a_body,
    out_type=jax.ShapeDtypeStruct((NC, N, D), jnp.float32),
    mesh=_mesh,
    scratch_types=[
        pltpu.VMEM_SHARED((N, D), jnp.float32),
        pltpu.VMEM((CHUNK,), jnp.int32),
        pltpu.VMEM((CHUNK,), jnp.int32),
        pltpu.VMEM((CHUNK,), jnp.int32),
        pltpu.VMEM((CHUNK,), jnp.int32),
        pltpu.VMEM((CHUNK,), jnp.float32),
        pltpu.VMEM((CHUNK,), jnp.float32),
        pltpu.VMEM((CHUNK, D), jnp.float32),
        pltpu.VMEM((CHUNK, D), jnp.float32),
        pltpu.SemaphoreType.DMA,
        pltpu.SemaphoreType.DMA,
    ],
)


_GRP = 5
_GE = _GRP * CHUNK
_NGRP = EPW // _GE


def _p1b_body(dst4_hbm, smsg_hbm, zb_hbm,
              bpart_hbm,
              b_sh, dst_v, mrow_v, sem):
    cid = lax.axis_index("c")
    sid = lax.axis_index("s")
    wid = cid * NS + sid
    base = wid * EPW

    @pl.when(sid == 0)
    def _():
        pltpu.sync_copy(zb_hbm, b_sh)
    plsc.subcore_barrier()

    def grp_body(i, carry):
        off = base + i * _GE
        pltpu.sync_copy(dst4_hbm.at[wid].at[i], dst_v)
        pltpu.sync_copy(smsg_hbm.at[pl.ds(off, _GE), :], mrow_v)
        cps = [pltpu.async_copy(mrow_v.at[pl.ds(j * CHUNK, CHUNK), :],
                                b_sh.at[dst_v.at[j]], sem, add=True)
               for j in range(_GRP)]
        for cp in cps:
            cp.wait()
        return carry
    lax.fori_loop(0, _NGRP, grp_body, 0)

    plsc.subcore_barrier()
    _copy_out(b_sh, bpart_hbm, cid, sid)


_p1b = pl.kernel(
    _p1b_body,
    out_type=jax.ShapeDtypeStruct((NC, N, DE), jnp.float32),
    mesh=_mesh,
    scratch_types=[
        pltpu.VMEM_SHARED((N, DE), jnp.float32),
        pltpu.VMEM((_GRP, CHUNK), jnp.int32),
        pltpu.VMEM((_GE, DE), jnp.float32),
        pltpu.SemaphoreType.DMA,
    ],
)


_SB = 8000

def _smsg_body(t_ref, msg_ref, out_ref):
    out_ref[...] = msg_ref[...] * jax.nn.sigmoid(t_ref[...])


def _smsg_call(t, msg):
    return pl.pallas_call(
        _smsg_body,
        grid=(E // _SB,),
        in_specs=[
            pl.BlockSpec((_SB, 1), lambda i: (i, 0)),
            pl.BlockSpec((_SB, DE), lambda i: (i, 0)),
        ],
        out_specs=pl.BlockSpec((_SB, DE), lambda i: (i, 0)),
        out_shape=jax.ShapeDtypeStruct((E, DE), jnp.float32),
    )(t.reshape(E, 1), msg)


def _h_body(ap_ref, bp_ref, x_ref, wmsg_ref, wself_ref, wupd_ref, h_ref):
    a = ap_ref[0] + ap_ref[1]
    b = bp_ref[0] + bp_ref[1]
    agg = jnp.dot(a, wmsg_ref[0:D, :], preferred_element_type=jnp.float32)
    agg += jnp.dot(b, wmsg_ref[D:DAB, :], preferred_element_type=jnp.float32)
    acc = jnp.dot(agg, wupd_ref[...], preferred_element_type=jnp.float32)
    acc += jnp.dot(x_ref[...], wself_ref[...], preferred_element_type=jnp.float32)
    h_ref[...] = jnp.maximum(acc, 0.0)


_HR = 1000

def _h_call(apart, bpart, x, w_msg, w_self, w_upd):
    return pl.pallas_call(
        _h_body,
        grid=(N // _HR,),
        in_specs=[
            pl.BlockSpec((NC, _HR, D), lambda i: (0, i, 0)),
            pl.BlockSpec((NC, _HR, DE), lambda i: (0, i, 0)),
            pl.BlockSpec((_HR, D), lambda i: (i, 0)),
            pl.BlockSpec((DAB, D), lambda i: (0, 0)),
            pl.BlockSpec((D, D), lambda i: (0, 0)),
            pl.BlockSpec((D, D), lambda i: (0, 0)),
        ],
        out_specs=pl.BlockSpec((_HR, D), lambda i: (i, 0)),
        out_shape=jax.ShapeDtypeStruct((N, D), jnp.float32),
    )(apart, bpart, x, w_msg, w_self, w_upd)


def _p3_body(h_hbm, src_hbm, dst_hbm, pd_hbm,
             src0_v, src1_v, dst0_v, dst1_v,
             hs0_v, hs1_v, hd0_v, hd1_v, pd_v, sem0, sem1):
    cid = lax.axis_index("c")
    sid = lax.axis_index("s")
    wid = cid * NS + sid
    base = wid * EPW

    def load_and_gather(c, src_v, dst_v, hs_v, hd_v, sem):
        off = base + c * CHUNK
        pltpu.sync_copy(src_hbm.at[pl.ds(off, CHUNK)], src_v)
        pltpu.sync_copy(dst_hbm.at[pl.ds(off, CHUNK)], dst_v)
        pltpu.async_copy(h_hbm.at[src_v], hs_v, sem)
        pltpu.async_copy(h_hbm.at[dst_v], hd_v, sem)

    def drain_gather(src_v, dst_v, hs_v, hd_v, sem):
        pltpu.make_async_copy(h_hbm.at[src_v], hs_v, sem).wait()
        pltpu.make_async_copy(h_hbm.at[dst_v], hd_v, sem).wait()

    def dot_and_store(c, hs_v, hd_v):
        def edge_body(e, carry2):
            acc = hs_v[e, pl.ds(0, LANES)] * hd_v[e, pl.ds(0, LANES)]
            for j in range(1, D // LANES):
                sl = pl.ds(j * LANES, LANES)
                acc = acc + hs_v[e, sl] * hd_v[e, sl]
            pd_v[e, :] = acc
            return carry2
        lax.fori_loop(0, CHUNK, edge_body, 0)
        off = base + c * CHUNK
        pltpu.sync_copy(pd_v, pd_hbm.at[pl.ds(off, CHUNK), :])

    load_and_gather(0, src0_v, dst0_v, hs0_v, hd0_v, sem0)

    def pair_body(i, carry):
        c0 = 2 * i
        load_and_gather(c0 + 1, src1_v, dst1_v, hs1_v, hd1_v, sem1)
        drain_gather(src0_v, dst0_v, hs0_v, hd0_v, sem0)
        dot_and_store(c0, hs0_v, hd0_v)
        load_and_gather(c0 + 2, src0_v, dst0_v, hs0_v, hd0_v, sem0)
        drain_gather(src1_v, dst1_v, hs1_v, hd1_v, sem1)
        dot_and_store(c0 + 1, hs1_v, hd1_v)
        return carry
    lax.fori_loop(0, (NCHUNK - 1) // 2, pair_body, 0)

    drain_gather(src0_v, dst0_v, hs0_v, hd0_v, sem0)
    dot_and_store(NCHUNK - 1, hs0_v, hd0_v)


_p3 = pl.kernel(
    _p3_body,
    out_type=jax.ShapeDtypeStruct((E, LANES), jnp.float32),
    mesh=_mesh,
    scratch_types=[
        pltpu.VMEM((CHUNK,), jnp.int32),
        pltpu.VMEM((CHUNK,), jnp.int32),
        pltpu.VMEM((CHUNK,), jnp.int32),
        pltpu.VMEM((CHUNK,), jnp.int32),
        pltpu.VMEM((CHUNK, D), jnp.float32),
        pltpu.VMEM((CHUNK, D), jnp.float32),
        pltpu.VMEM((CHUNK, D), jnp.float32),
        pltpu.VMEM((CHUNK, D), jnp.float32),
        pltpu.VMEM((CHUNK, LANES), jnp.float32),
        pltpu.SemaphoreType.DMA,
        pltpu.SemaphoreType.DMA,
    ],
)


_EB = 8000

def _loss_body(pd_ref, out_ref):
    i = pl.program_id(0)
    logits = jnp.sum(pd_ref[...], axis=1)
    part = jnp.sum(jax.nn.softplus(-logits)) * (1.0 / E)

    @pl.when(i == 0)
    def _():
        out_ref[0, 0] = part

    @pl.when(i > 0)
    def _():
        out_ref[0, 0] += part


def _loss_call(pdot):
    return pl.pallas_call(
        _loss_body,
        grid=(E // _EB,),
        in_specs=[pl.BlockSpec((_EB, LANES), lambda i: (i, 0))],
        out_specs=pl.BlockSpec(memory_space=pltpu.SMEM),
        out_shape=jax.ShapeDtypeStruct((1, 1), jnp.float32),
    )(pdot)


def kernel(x, edge_index, t, msg, edge_type, W_msg, W_self, W_upd):
    del edge_type
    src = edge_index[0]
    dst = edge_index[1]
    za = jnp.zeros((N, D), jnp.float32)
    zb = jnp.zeros((N, DE), jnp.float32)
    apart = _p1a(x, src, dst, t, za)
    dst4 = dst.reshape(NW, _NGRP, _GRP, CHUNK)
    bpart = _p1b(dst4, _smsg_call(t, msg), zb)
    h = _h_call(apart, bpart, x, W_msg, W_self, W_upd)
    pdot = _p3(h, src, dst)
    loss = _loss_call(pdot)
    return loss.reshape((1,))
